# baseline (speedup 1.0000x reference)
import functools

import jax
import jax.numpy as jnp
from jax import lax
from jax.experimental import pallas as pl
from jax.experimental.pallas import tpu as pltpu
from jax.experimental.pallas import tpu_sc as plsc

_ROW_STRIDE = 8192


def _gather_kernel(x_hbm, out_hbm, buf, obuf):
    wid = lax.axis_index("s") * 2 + lax.axis_index("c")

    @pl.when(wid == 0)
    def _():
        pltpu.sync_copy(x_hbm.at[pl.ds(0, 16)], buf.at[pl.ds(0, 16)])
        pltpu.sync_copy(x_hbm.at[pl.ds(_ROW_STRIDE, 16)], buf.at[pl.ds(16, 16)])
        i = lax.iota(jnp.int32, 16)
        zero = jnp.zeros((16,), jnp.int32)
        one = jnp.ones((16,), jnp.int32)
        row = jnp.maximum(zero, jnp.minimum(one, i - 2))
        col = jnp.maximum(zero, jnp.minimum(one, i)) - jnp.maximum(
            zero, jnp.minimum(one, i - 3))
        obuf[...] = plsc.load_gather(buf, [row * 16 + col])
        pltpu.sync_copy(obuf, out_hbm)


def kernel(x):
    xf = x.reshape(-1)
    mesh = plsc.VectorSubcoreMesh(core_axis_name="c", subcore_axis_name="s")
    run = functools.partial(
        pl.kernel,
        mesh=mesh,
        compiler_params=pltpu.CompilerParams(needs_layout_passes=False),
        out_type=jax.ShapeDtypeStruct((16,), jnp.float32),
        scratch_types=[
            pltpu.VMEM((32,), jnp.float32),
            pltpu.VMEM((16,), jnp.float32),
        ],
    )(_gather_kernel)
    out = run(xf)
    return out[:6].reshape(2, 3)


# speedup vs baseline: 1.1047x; 1.1047x over previous
import functools

import jax
import jax.numpy as jnp
from jax import lax
from jax.experimental import pallas as pl
from jax.experimental.pallas import tpu as pltpu
from jax.experimental.pallas import tpu_sc as plsc

_ROW_STRIDE = 8192


def _gather_kernel(x_hbm, out_hbm, gbuf, sem):
    wid = lax.axis_index("s") * 2 + lax.axis_index("c")

    @pl.when(wid == 0)
    def _():
        i = lax.iota(jnp.int32, 16)
        zero = jnp.zeros((16,), jnp.int32)
        one = jnp.ones((16,), jnp.int32)
        row = jnp.maximum(zero, jnp.minimum(one, i - 2))
        col = jnp.maximum(zero, jnp.minimum(one, i)) - jnp.maximum(
            zero, jnp.minimum(one, i - 3))
        idx = row * _ROW_STRIDE + col
        pltpu.async_copy(x_hbm.at[idx], gbuf, sem).wait()
        pltpu.sync_copy(gbuf.at[pl.ds(0, 6)], out_hbm)


def kernel(x):
    xf = x.reshape(-1)
    mesh = plsc.VectorSubcoreMesh(
        core_axis_name="c", subcore_axis_name="s", num_cores=1)
    run = functools.partial(
        pl.kernel,
        mesh=mesh,
        compiler_params=pltpu.CompilerParams(needs_layout_passes=False),
        out_type=jax.ShapeDtypeStruct((6,), jnp.float32),
        scratch_types=[
            pltpu.VMEM((16,), jnp.float32),
            pltpu.SemaphoreType.DMA,
        ],
    )(_gather_kernel)
    return run(xf).reshape(2, 3)
